# baseline (speedup 1.0000x reference)
import functools

import jax
import jax.numpy as jnp
from jax import lax
from jax.experimental import pallas as pl
from jax.experimental.pallas import tpu as pltpu
from jax.experimental.pallas import tpu_sc as plsc

_D = 64
_NSLOT = 3
_NJ = 200
_NJT = _NJ // 8
_NIT = 128


@functools.lru_cache(maxsize=None)
def _build():
    info = plsc.get_sparse_core_info()
    nw = info.num_cores * info.num_subcores
    it_per_w = _NIT // nw

    mesh = plsc.VectorSubcoreMesh(core_axis_name="c", subcore_axis_name="s")

    @functools.partial(
        pl.kernel,
        out_type=jax.ShapeDtypeStruct((_NJ, 8, _NIT, 8, 128), jnp.float32),
        mesh=mesh,
        compiler_params=pltpu.CompilerParams(
            use_tc_tiling_on_sc=False, needs_layout_passes=False
        ),
        scratch_types=[
            pltpu.VMEM((_NJT, 8, 128), jnp.int32),
            [pltpu.VMEM((128, _D), jnp.float32)] * _NSLOT,
            [pltpu.VMEM((8, 8, 128), jnp.float32)] * _NSLOT,
            [pltpu.SemaphoreType.DMA] * _NSLOT,
            [pltpu.SemaphoreType.DMA] * _NSLOT,
        ],
    )
    def emb_lookup(x4_hbm, emb_hbm, out_hbm, idx_s, rows_s, wb_s, gsems,
                   wsems):
        wid = lax.axis_index("s") * info.num_cores + lax.axis_index("c")
        iota = lax.iota(jnp.int32, 16)
        gidx = [iota + 16 * g for g in range(8)]

        def start(j, slot):
            pltpu.async_copy(
                emb_hbm.at[idx_s.at[j // 8, j % 8]], rows_s[slot],
                gsems[slot],
            )

        def finish(it, j, slot):
            pltpu.make_async_copy(
                emb_hbm.at[pl.ds(0, 128)], rows_s[slot], gsems[slot]
            ).wait()

            @pl.loop(0, 8)
            def _dt(dt):
                for ds in range(8):
                    dsp = jnp.full((16,), dt * 8 + ds, jnp.int32)
                    for g in range(8):
                        v = plsc.load_gather(rows_s[slot], [gidx[g], dsp])
                        wb_s[slot][dt, ds, pl.ds(16 * g, 16)] = v

            pltpu.async_copy(
                wb_s[slot], out_hbm.at[j, pl.ds(0, 8), it], wsems[slot]
            )

        def wait_wb(slot):
            pltpu.make_async_copy(
                wb_s[slot], out_hbm.at[0, pl.ds(0, 8), 0], wsems[slot]
            ).wait()

        @pl.loop(0, it_per_w)
        def _itb(b):
            it = wid * it_per_w + b
            pltpu.sync_copy(x4_hbm.at[pl.ds(0, _NJT), it], idx_s)

            start(0, 0)

            @pl.loop(0, _NJ, step=_NSLOT)
            def _triplet(j):
                @pl.when(jnp.logical_and(j > 0, j + 1 < _NJ))
                def _():
                    wait_wb(1)

                @pl.when(j + 1 < _NJ)
                def _():
                    start(j + 1, 1)

                @pl.when(jnp.logical_and(j > 0, j + 2 < _NJ))
                def _():
                    wait_wb(2)

                @pl.when(j + 2 < _NJ)
                def _():
                    start(j + 2, 2)

                finish(it, j, 0)

                @pl.when(j + 1 < _NJ)
                def _():
                    finish(it, j + 1, 1)

                @pl.when(j + 2 < _NJ)
                def _():
                    finish(it, j + 2, 2)

                @pl.when(j + _NSLOT < _NJ)
                def _():
                    wait_wb(0)
                    start(j + _NSLOT, 0)

            for s in range(_NSLOT):
                wait_wb(s)

    return emb_lookup


def kernel(x, emb):
    x4 = x.astype(jnp.int32).reshape(128, 128, 25, 8).transpose(2, 0, 3, 1)
    out5 = _build()(x4, emb)
    return out5.transpose(2, 4, 0, 1, 3).reshape(16384, 200, 64)
